# Initial kernel scaffold; baseline (speedup 1.0000x reference)
import jax
import jax.numpy as jnp
from jax.experimental import pallas as pl


def kernel(x, pos, tq, energy, batch, W1, b1, W2, b2, W3, b3, W4, b4, W5, b5, W6, b6, W7, b7, W8, b8):
    raise NotImplementedError("write your pallas kernel here")


# speedup vs baseline: 7.0676x; 7.0676x over previous
import jax
import jax.numpy as jnp
from jax.experimental import pallas as pl
from jax.experimental.pallas import tpu as pltpu

_B = 500
_NPG = 100
_K = 16


def _lrelu(v):
    return jnp.where(v >= 0, v, 0.01 * v)


def _edcn_program(pos_ref, posT_ref, xx_ref, energy_ref,
                  w1d_ref, w1b_ref, b1_ref, w2_ref, b2_ref,
                  w3d_ref, w3b_ref, b3_ref, w4_ref, b4_ref,
                  w5_ref, b5_ref, w6_ref, b6_ref,
                  w7_ref, b7_ref, w8_ref, b8_ref,
                  out_ref):
    pos = pos_ref[0]
    posT = posT_ref[0]
    xx = xx_ref[0]

    d2 = (pos[:, 0:1] - posT[0:1, :]) ** 2
    d2 = d2 + (pos[:, 1:2] - posT[1:2, :]) ** 2
    d2 = d2 + (pos[:, 2:3] - posT[2:3, :]) ** 2

    jidx = jax.lax.broadcasted_iota(
        jnp.int32, (_NPG, _NPG), 1).astype(jnp.float32)

    ohs = []
    d = d2
    for _ in range(_K):
        m = jnp.min(d, axis=1, keepdims=True)
        cand = jnp.where(d == m, jidx, jnp.float32(1e9))
        jstar = jnp.min(cand, axis=1, keepdims=True)
        sel = jidx == jstar
        ohs.append(sel.astype(jnp.float32))
        d = jnp.where(sel, jnp.float32(jnp.inf), d)

    def conv(feat, wd_ref, wb_ref, bin_ref, wout_ref, bout_ref, act):
        wd = wd_ref[...]
        wb = wb_ref[...]
        p = jnp.dot(feat, wd, preferred_element_type=jnp.float32) + bin_ref[...]
        q = jnp.dot(feat, wb, preferred_element_type=jnp.float32)
        w2 = wout_ref[...]
        acc = None
        for oh in ohs:
            qk = jnp.dot(oh, q, preferred_element_type=jnp.float32)
            hk = act(p + qk)
            zk = jnp.dot(hk, w2, preferred_element_type=jnp.float32)
            acc = zk if acc is None else jnp.maximum(acc, zk)
        return act(acc + bout_ref[...])

    x1 = conv(xx, w1d_ref, w1b_ref, b1_ref, w2_ref, b2_ref,
              lambda v: jnp.maximum(v, 0.0))
    x2 = conv(x1, w3d_ref, w3b_ref, b3_ref, w4_ref, b4_ref, _lrelu)
    x3 = conv(x2, w3d_ref, w3b_ref, b3_ref, w4_ref, b4_ref, _lrelu)

    comb = jnp.concatenate([x1, x2, x3], axis=1)
    h = jnp.maximum(
        jnp.dot(comb, w5_ref[...], preferred_element_type=jnp.float32)
        + b5_ref[...], 0.0)
    h = jnp.dot(h, w6_ref[...], preferred_element_type=jnp.float32) + b6_ref[...]

    pooled = jnp.sum(h, axis=0, keepdims=True) / jnp.float32(_NPG)
    og = jnp.concatenate([pooled, energy_ref[0]], axis=1)
    og = jnp.maximum(og, 0.0)
    o = jnp.maximum(
        jnp.dot(og, w7_ref[...], preferred_element_type=jnp.float32)
        + b7_ref[...], 0.0)
    res = jnp.dot(o, w8_ref[...], preferred_element_type=jnp.float32) + b8_ref[...]
    out_ref[0] = res


def _full(shape):
    nd = len(shape)
    return pl.BlockSpec(shape, lambda g: (0,) * nd)


def kernel(x, pos, tq, energy, batch,
           W1, b1, W2, b2, W3, b3, W4, b4, W5, b5, W6, b6, W7, b7, W8, b8):
    xx = jnp.concatenate([tq, x, pos], axis=1)
    posb = pos.reshape(_B, _NPG, 3)
    posTb = posb.transpose(0, 2, 1)
    xxb = xx.reshape(_B, _NPG, 5)
    energyb = energy.reshape(_B, 1, 1)
    w1d = W1[:5] - W1[5:]
    w1b = W1[5:]
    w3d = W3[:32] - W3[32:]
    w3b = W3[32:]
    b1r = b1.reshape(1, -1)
    b2r = b2.reshape(1, -1)
    b3r = b3.reshape(1, -1)
    b4r = b4.reshape(1, -1)
    b5r = b5.reshape(1, -1)
    b6r = b6.reshape(1, -1)
    b7r = b7.reshape(1, -1)
    b8r = b8.reshape(1, -1)

    out = pl.pallas_call(
        _edcn_program,
        grid=(_B,),
        in_specs=[
            pl.BlockSpec((1, _NPG, 3), lambda g: (g, 0, 0)),
            pl.BlockSpec((1, 3, _NPG), lambda g: (g, 0, 0)),
            pl.BlockSpec((1, _NPG, 5), lambda g: (g, 0, 0)),
            pl.BlockSpec((1, 1, 1), lambda g: (g, 0, 0)),
            _full(w1d.shape), _full(w1b.shape), _full(b1r.shape),
            _full(W2.shape), _full(b2r.shape),
            _full(w3d.shape), _full(w3b.shape), _full(b3r.shape),
            _full(W4.shape), _full(b4r.shape),
            _full(W5.shape), _full(b5r.shape),
            _full(W6.shape), _full(b6r.shape),
            _full(W7.shape), _full(b7r.shape),
            _full(W8.shape), _full(b8r.shape),
        ],
        out_specs=pl.BlockSpec((1, 1, 4), lambda g: (g, 0, 0)),
        out_shape=jax.ShapeDtypeStruct((_B, 1, 4), jnp.float32),
        compiler_params=pltpu.CompilerParams(
            dimension_semantics=("parallel",)),
    )(posb, posTb, xxb, energyb,
      w1d, w1b, b1r, W2, b2r, w3d, w3b, b3r, W4, b4r,
      W5, b5r, W6, b6r, W7, b7r, W8, b8r)
    return out.reshape(_B, 4)


# speedup vs baseline: 12.5842x; 1.7806x over previous
import jax
import jax.numpy as jnp
from jax.experimental import pallas as pl
from jax.experimental.pallas import tpu as pltpu

_B = 500
_NPG = 100
_K = 16
_G = 4

_bf = lambda v: v.astype(jnp.bfloat16)


def _lrelu(v):
    return jnp.maximum(v, 0.01 * v)


def _relu(v):
    return jnp.maximum(v, 0.0)


def _dotbf(xbf, wbf):
    return jnp.dot(xbf, wbf, preferred_element_type=jnp.float32)


def _edcn_program(pos_ref, posT_ref, xx_ref, energy_ref,
                  wpq1_ref, b1_ref, w2_ref, b2_ref,
                  wpq3_ref, b3_ref, w4_ref, b4_ref,
                  w5_ref, b5_ref, w6_ref, b6_ref,
                  w7_ref, b7_ref, w8_ref, b8_ref,
                  out_ref):
    G = range(_G)
    wpq1 = wpq1_ref[...]
    b1 = b1_ref[...]
    w2 = w2_ref[...]
    b2 = b2_ref[...]
    wpq3 = wpq3_ref[...]
    b3 = b3_ref[...]
    w4 = w4_ref[...]
    b4 = b4_ref[...]

    ds = []
    for s in G:
        pos = pos_ref[0, s]
        posT = posT_ref[0, s]
        d2 = (pos[:, 0:1] - posT[0:1, :]) ** 2
        d2 = d2 + (pos[:, 1:2] - posT[1:2, :]) ** 2
        d2 = d2 + (pos[:, 2:3] - posT[2:3, :]) ** 2
        ds.append(d2)

    jidx = jax.lax.broadcasted_iota(
        jnp.int32, (_NPG, _NPG), 1).astype(jnp.float32)

    ohs = [[] for _ in G]
    for _ in range(_K):
        for s in G:
            d = ds[s]
            m = jnp.min(d, axis=1, keepdims=True)
            cand = jnp.where(d == m, jidx, jnp.float32(1e9))
            jstar = jnp.min(cand, axis=1, keepdims=True)
            sel = jidx == jstar
            ohs[s].append(sel.astype(jnp.bfloat16))
            ds[s] = jnp.where(sel, jnp.float32(jnp.inf), d)

    def conv(featbfs, wpq, bin_, wout, bout, act):
        f = bin_.shape[-1]
        ps, qs = [], []
        for s in G:
            pq = _dotbf(featbfs[s], wpq)
            ps.append(pq[:, :f] + bin_)
            qs.append(_bf(pq[:, f:]))
        accs = [None] * _G
        for k in range(_K):
            for s in G:
                qk = _dotbf(ohs[s][k], qs[s])
                hk = act(ps[s] + qk)
                zk = _dotbf(_bf(hk), wout)
                accs[s] = zk if accs[s] is None else jnp.maximum(accs[s], zk)
        return [act(accs[s] + bout) for s in G]

    xxbf = [_bf(xx_ref[0, s]) for s in G]
    x1 = conv(xxbf, wpq1, b1, w2, b2, _relu)
    x2 = conv([_bf(v) for v in x1], wpq3, b3, w4, b4, _lrelu)
    x3 = conv([_bf(v) for v in x2], wpq3, b3, w4, b4, _lrelu)

    w5 = w5_ref[...]
    b5 = b5_ref[...]
    w6 = w6_ref[...]
    b6 = b6_ref[...]
    w7 = w7_ref[...]
    b7 = b7_ref[...]
    w8 = w8_ref[...]
    b8 = b8_ref[...]
    for s in G:
        comb = jnp.concatenate([_bf(x1[s]), _bf(x2[s]), _bf(x3[s])], axis=1)
        h = _relu(_dotbf(comb, w5) + b5)
        h = _dotbf(_bf(h), w6) + b6
        pooled = jnp.sum(h, axis=0, keepdims=True) / jnp.float32(_NPG)
        e = energy_ref[0, 0, 0, s]
        og = jnp.concatenate(
            [pooled, jnp.full((1, 1), e, jnp.float32)], axis=1)
        og = _relu(og)
        o = _relu(jnp.dot(og, w7, preferred_element_type=jnp.float32) + b7)
        res = jnp.dot(o, w8, preferred_element_type=jnp.float32) + b8
        out_ref[0, 0, s] = res[0]


def _full(shape):
    nd = len(shape)
    return pl.BlockSpec(shape, lambda g: (0,) * nd)


def kernel(x, pos, tq, energy, batch,
           W1, b1, W2, b2, W3, b3, W4, b4, W5, b5, W6, b6, W7, b7, W8, b8):
    nb = _B // _G
    xx = jnp.concatenate([tq, x, pos], axis=1)
    posb = pos.reshape(nb, _G, _NPG, 3)
    posTb = posb.transpose(0, 1, 3, 2)
    xxb = xx.reshape(nb, _G, _NPG, 5)
    energyb = energy.reshape(nb, 1, 1, _G)
    wpq1 = _bf(jnp.concatenate([W1[:5] - W1[5:], W1[5:]], axis=1))
    wpq3 = _bf(jnp.concatenate([W3[:32] - W3[32:], W3[32:]], axis=1))
    w2b = _bf(W2)
    w4b = _bf(W4)
    w5b = _bf(W5)
    w6b = _bf(W6)
    b1r = b1.reshape(1, -1)
    b2r = b2.reshape(1, -1)
    b3r = b3.reshape(1, -1)
    b4r = b4.reshape(1, -1)
    b5r = b5.reshape(1, -1)
    b6r = b6.reshape(1, -1)
    b7r = b7.reshape(1, -1)
    b8r = b8.reshape(1, -1)

    out = pl.pallas_call(
        _edcn_program,
        grid=(nb,),
        in_specs=[
            pl.BlockSpec((1, _G, _NPG, 3), lambda g: (g, 0, 0, 0)),
            pl.BlockSpec((1, _G, 3, _NPG), lambda g: (g, 0, 0, 0)),
            pl.BlockSpec((1, _G, _NPG, 5), lambda g: (g, 0, 0, 0)),
            pl.BlockSpec((1, 1, 1, _G), lambda g: (g, 0, 0, 0)),
            _full(wpq1.shape), _full(b1r.shape),
            _full(w2b.shape), _full(b2r.shape),
            _full(wpq3.shape), _full(b3r.shape),
            _full(w4b.shape), _full(b4r.shape),
            _full(w5b.shape), _full(b5r.shape),
            _full(w6b.shape), _full(b6r.shape),
            _full(W7.shape), _full(b7r.shape),
            _full(W8.shape), _full(b8r.shape),
        ],
        out_specs=pl.BlockSpec((1, 1, _G, 4), lambda g: (g, 0, 0, 0)),
        out_shape=jax.ShapeDtypeStruct((nb, 1, _G, 4), jnp.float32),
        compiler_params=pltpu.CompilerParams(
            dimension_semantics=("parallel",)),
    )(posb, posTb, xxb, energyb,
      wpq1, b1r, w2b, b2r, wpq3, b3r, w4b, b4r,
      w5b, b5r, w6b, b6r, W7, b7r, W8, b8r)
    return out.reshape(_B, 4)


# speedup vs baseline: 13.1210x; 1.0427x over previous
import jax
import jax.numpy as jnp
from jax.experimental import pallas as pl
from jax.experimental.pallas import tpu as pltpu

_B = 500
_NPG = 100
_K = 16
_G = 10

_bf = lambda v: v.astype(jnp.bfloat16)


def _lrelu(v):
    return jnp.maximum(v, 0.01 * v)


def _relu(v):
    return jnp.maximum(v, 0.0)


def _dotbf(xbf, wbf):
    return jnp.dot(xbf, wbf, preferred_element_type=jnp.float32)


def _edcn_program(pos_ref, posT_ref, xx_ref, energy_ref,
                  wpq1_ref, b1_ref, w2_ref, b2_ref,
                  wpq3_ref, b3_ref, w4_ref, b4_ref,
                  w5_ref, b5_ref, w6_ref, b6_ref,
                  w7_ref, b7_ref, w8_ref, b8_ref,
                  out_ref):
    G = range(_G)
    wpq1 = wpq1_ref[...]
    b1 = b1_ref[...]
    w2 = w2_ref[...]
    b2 = b2_ref[...]
    wpq3 = wpq3_ref[...]
    b3 = b3_ref[...]
    w4 = w4_ref[...]
    b4 = b4_ref[...]

    ds = []
    for s in G:
        pos = pos_ref[0, s]
        posT = posT_ref[0, s]
        d2 = (pos[:, 0:1] - posT[0:1, :]) ** 2
        d2 = d2 + (pos[:, 1:2] - posT[1:2, :]) ** 2
        d2 = d2 + (pos[:, 2:3] - posT[2:3, :]) ** 2
        ds.append(d2)

    jidx = jax.lax.broadcasted_iota(
        jnp.int32, (_NPG, _NPG), 1).astype(jnp.float32)

    ohs = [[] for _ in G]
    for _ in range(_K):
        for s in G:
            d = ds[s]
            m = jnp.min(d, axis=1, keepdims=True)
            cand = jnp.where(d == m, jidx, jnp.float32(1e9))
            jstar = jnp.min(cand, axis=1, keepdims=True)
            sel = jidx == jstar
            ohs[s].append(sel.astype(jnp.bfloat16))
            ds[s] = jnp.where(sel, jnp.float32(jnp.inf), d)

    def conv(featbfs, wpq, bin_, wout, bout, act):
        f = bin_.shape[-1]
        ps, qs = [], []
        for s in G:
            pq = _dotbf(featbfs[s], wpq)
            ps.append(pq[:, :f] + bin_)
            qs.append(_bf(pq[:, f:]))
        accs = [None] * _G
        for k in range(_K):
            for s in G:
                qk = _dotbf(ohs[s][k], qs[s])
                hk = act(ps[s] + qk)
                zk = _dotbf(_bf(hk), wout)
                accs[s] = zk if accs[s] is None else jnp.maximum(accs[s], zk)
        return [act(accs[s] + bout) for s in G]

    xxbf = [_bf(xx_ref[0, s]) for s in G]
    x1 = conv(xxbf, wpq1, b1, w2, b2, _relu)
    x2 = conv([_bf(v) for v in x1], wpq3, b3, w4, b4, _lrelu)
    x3 = conv([_bf(v) for v in x2], wpq3, b3, w4, b4, _lrelu)

    w5 = w5_ref[...]
    b5 = b5_ref[...]
    w6 = w6_ref[...]
    b6 = b6_ref[...]
    w7 = w7_ref[...]
    b7 = b7_ref[...]
    w8 = w8_ref[...]
    b8 = b8_ref[...]
    for s in G:
        comb = jnp.concatenate([_bf(x1[s]), _bf(x2[s]), _bf(x3[s])], axis=1)
        h = _relu(_dotbf(comb, w5) + b5)
        h = _dotbf(_bf(h), w6) + b6
        pooled = jnp.sum(h, axis=0, keepdims=True) / jnp.float32(_NPG)
        e = energy_ref[0, 0, 0, s]
        og = jnp.concatenate(
            [pooled, jnp.full((1, 1), e, jnp.float32)], axis=1)
        og = _relu(og)
        o = _relu(jnp.dot(og, w7, preferred_element_type=jnp.float32) + b7)
        res = jnp.dot(o, w8, preferred_element_type=jnp.float32) + b8
        out_ref[0, 0, s] = res[0]


def _full(shape):
    nd = len(shape)
    return pl.BlockSpec(shape, lambda g: (0,) * nd)


def kernel(x, pos, tq, energy, batch,
           W1, b1, W2, b2, W3, b3, W4, b4, W5, b5, W6, b6, W7, b7, W8, b8):
    nb = _B // _G
    xx = jnp.concatenate([tq, x, pos], axis=1)
    posb = pos.reshape(nb, _G, _NPG, 3)
    posTb = posb.transpose(0, 1, 3, 2)
    xxb = xx.reshape(nb, _G, _NPG, 5)
    energyb = energy.reshape(nb, 1, 1, _G)
    wpq1 = _bf(jnp.concatenate([W1[:5] - W1[5:], W1[5:]], axis=1))
    wpq3 = _bf(jnp.concatenate([W3[:32] - W3[32:], W3[32:]], axis=1))

    w2b = _bf(W2)
    w4b = _bf(W4)
    w5b = _bf(W5)
    w6b = _bf(W6)
    b1r = b1.reshape(1, -1)
    b2r = b2.reshape(1, -1)
    b3r = b3.reshape(1, -1)
    b4r = b4.reshape(1, -1)
    b5r = b5.reshape(1, -1)
    b6r = b6.reshape(1, -1)
    b7r = b7.reshape(1, -1)
    b8r = b8.reshape(1, -1)

    out = pl.pallas_call(
        _edcn_program,
        grid=(nb,),
        in_specs=[
            pl.BlockSpec((1, _G, _NPG, 3), lambda g: (g, 0, 0, 0)),
            pl.BlockSpec((1, _G, 3, _NPG), lambda g: (g, 0, 0, 0)),
            pl.BlockSpec((1, _G, _NPG, 5), lambda g: (g, 0, 0, 0)),
            pl.BlockSpec((1, 1, 1, _G), lambda g: (g, 0, 0, 0)),
            _full(wpq1.shape), _full(b1r.shape),
            _full(w2b.shape), _full(b2r.shape),
            _full(wpq3.shape), _full(b3r.shape),
            _full(w4b.shape), _full(b4r.shape),
            _full(w5b.shape), _full(b5r.shape),
            _full(w6b.shape), _full(b6r.shape),
            _full(W7.shape), _full(b7r.shape),
            _full(W8.shape), _full(b8r.shape),
        ],
        out_specs=pl.BlockSpec((1, 1, _G, 4), lambda g: (g, 0, 0, 0)),
        out_shape=jax.ShapeDtypeStruct((nb, 1, _G, 4), jnp.float32),
        compiler_params=pltpu.CompilerParams(
            dimension_semantics=("parallel",)),
    )(posb, posTb, xxb, energyb,
      wpq1, b1r, w2b, b2r, wpq3, b3r, w4b, b4r,
      w5b, b5r, w6b, b6r, W7, b7r, W8, b8r)
    return out.reshape(_B, 4)


# speedup vs baseline: 13.5983x; 1.0364x over previous
import jax
import jax.numpy as jnp
from jax.experimental import pallas as pl
from jax.experimental.pallas import tpu as pltpu

_B = 500
_NPG = 100
_K = 16
_G = 10

_bf = lambda v: v.astype(jnp.bfloat16)


def _lrelu(v):
    return jnp.maximum(v, 0.01 * v)


def _relu(v):
    return jnp.maximum(v, 0.0)


def _dotbf(xbf, wbf):
    return jnp.dot(xbf, wbf, preferred_element_type=jnp.float32)


def _edcn_program(pos_ref, posT_ref, xx_ref, energy_ref,
                  wpq1_ref, b1_ref, w2_ref, b2_ref,
                  wpq3_ref, b3_ref, w4_ref, b4_ref,
                  w5_ref, b5_ref, w6_ref, b6_ref,
                  w7_ref, b7_ref, w8_ref, b8_ref,
                  out_ref):
    G = range(_G)
    wpq1 = wpq1_ref[...]
    b1 = b1_ref[...]
    w2 = w2_ref[...]
    b2 = b2_ref[...]
    wpq3 = wpq3_ref[...]
    b3 = b3_ref[...]
    w4 = w4_ref[...]
    b4 = b4_ref[...]

    ds = []
    for s in G:
        pos = pos_ref[0, s]
        posT = posT_ref[0, s]
        d2 = (pos[:, 0:1] - posT[0:1, :]) ** 2
        d2 = d2 + (pos[:, 1:2] - posT[1:2, :]) ** 2
        d2 = d2 + (pos[:, 2:3] - posT[2:3, :]) ** 2
        ds.append(d2)

    jidx = jax.lax.broadcasted_iota(
        jnp.int32, (_NPG, _NPG), 1).astype(jnp.float32)

    ohs = [[] for _ in G]

    def topk_iter(s):
        d = ds[s]
        m = jnp.min(d, axis=1, keepdims=True)
        cand = jnp.where(d == m, jidx, jnp.float32(1e9))
        jstar = jnp.min(cand, axis=1, keepdims=True)
        sel = jidx == jstar
        ohs[s].append(sel.astype(jnp.bfloat16))
        ds[s] = jnp.where(sel, jnp.float32(jnp.inf), d)

    def conv(featbfs, subset, wpq, bin_, wout, bout, act, hook=None, nh=0):
        f = bin_.shape[-1]
        ps, qs = {}, {}
        for s in subset:
            pq = _dotbf(_bf(featbfs[s]), wpq)
            ps[s] = pq[:, :f] + bin_
            qs[s] = _bf(pq[:, f:])
        accs = {s: None for s in subset}
        for k in range(_K):
            for s in subset:
                qk = _dotbf(ohs[s][k], qs[s])
                hk = act(ps[s] + qk)
                zk = _dotbf(_bf(hk), wout)
                accs[s] = zk if accs[s] is None else jnp.maximum(accs[s], zk)
            if hook is not None:
                hook(nh)
        return {s: act(accs[s] + bout) for s in subset}

    half = _G // 2
    ga = list(range(half))
    gb = list(range(half, _G))
    for _ in range(_K):
        for s in ga:
            topk_iter(s)

    bqueue = [s for _ in range(_K) for s in gb]
    bpos = [0]

    def hook(n):
        for _ in range(n):
            if bpos[0] < len(bqueue):
                topk_iter(bqueue[bpos[0]])
                bpos[0] += 1

    xxbf = [_bf(xx_ref[0, s]) for s in G]
    x1 = conv(xxbf, ga, wpq1, b1, w2, b2, _relu, hook=hook, nh=3)
    x2 = conv(x1, ga, wpq3, b3, w4, b4, _lrelu, hook=hook, nh=2)
    hook(len(bqueue))
    x3 = conv(x2, ga, wpq3, b3, w4, b4, _lrelu)
    x1.update(conv(xxbf, gb, wpq1, b1, w2, b2, _relu))
    x2.update(conv(x1, gb, wpq3, b3, w4, b4, _lrelu))
    x3.update(conv(x2, gb, wpq3, b3, w4, b4, _lrelu))

    w5 = w5_ref[...]
    b5 = b5_ref[...]
    w6 = w6_ref[...]
    b6 = b6_ref[...]
    w7 = w7_ref[...]
    b7 = b7_ref[...]
    w8 = w8_ref[...]
    b8 = b8_ref[...]
    for s in G:
        comb = jnp.concatenate([_bf(x1[s]), _bf(x2[s]), _bf(x3[s])], axis=1)
        h = _relu(_dotbf(comb, w5) + b5)
        h = _dotbf(_bf(h), w6) + b6
        pooled = jnp.sum(h, axis=0, keepdims=True) / jnp.float32(_NPG)
        e = energy_ref[0, 0, 0, s]
        og = jnp.concatenate(
            [pooled, jnp.full((1, 1), e, jnp.float32)], axis=1)
        og = _relu(og)
        o = _relu(jnp.dot(og, w7, preferred_element_type=jnp.float32) + b7)
        res = jnp.dot(o, w8, preferred_element_type=jnp.float32) + b8
        out_ref[0, 0, s] = res[0]


def _full(shape):
    nd = len(shape)
    return pl.BlockSpec(shape, lambda g: (0,) * nd)


def kernel(x, pos, tq, energy, batch,
           W1, b1, W2, b2, W3, b3, W4, b4, W5, b5, W6, b6, W7, b7, W8, b8):
    nb = _B // _G
    xx = jnp.concatenate([tq, x, pos], axis=1)
    posb = pos.reshape(nb, _G, _NPG, 3)
    posTb = posb.transpose(0, 1, 3, 2)
    xxb = xx.reshape(nb, _G, _NPG, 5)
    energyb = energy.reshape(nb, 1, 1, _G)
    wpq1 = _bf(jnp.concatenate([W1[:5] - W1[5:], W1[5:]], axis=1))
    wpq3 = _bf(jnp.concatenate([W3[:32] - W3[32:], W3[32:]], axis=1))

    w2b = _bf(W2)
    w4b = _bf(W4)
    w5b = _bf(W5)
    w6b = _bf(W6)
    b1r = b1.reshape(1, -1)
    b2r = b2.reshape(1, -1)
    b3r = b3.reshape(1, -1)
    b4r = b4.reshape(1, -1)
    b5r = b5.reshape(1, -1)
    b6r = b6.reshape(1, -1)
    b7r = b7.reshape(1, -1)
    b8r = b8.reshape(1, -1)

    out = pl.pallas_call(
        _edcn_program,
        grid=(nb,),
        in_specs=[
            pl.BlockSpec((1, _G, _NPG, 3), lambda g: (g, 0, 0, 0)),
            pl.BlockSpec((1, _G, 3, _NPG), lambda g: (g, 0, 0, 0)),
            pl.BlockSpec((1, _G, _NPG, 5), lambda g: (g, 0, 0, 0)),
            pl.BlockSpec((1, 1, 1, _G), lambda g: (g, 0, 0, 0)),
            _full(wpq1.shape), _full(b1r.shape),
            _full(w2b.shape), _full(b2r.shape),
            _full(wpq3.shape), _full(b3r.shape),
            _full(w4b.shape), _full(b4r.shape),
            _full(w5b.shape), _full(b5r.shape),
            _full(w6b.shape), _full(b6r.shape),
            _full(W7.shape), _full(b7r.shape),
            _full(W8.shape), _full(b8r.shape),
        ],
        out_specs=pl.BlockSpec((1, 1, _G, 4), lambda g: (g, 0, 0, 0)),
        out_shape=jax.ShapeDtypeStruct((nb, 1, _G, 4), jnp.float32),
        compiler_params=pltpu.CompilerParams(
            dimension_semantics=("parallel",)),
    )(posb, posTb, xxb, energyb,
      wpq1, b1r, w2b, b2r, wpq3, b3r, w4b, b4r,
      w5b, b5r, w6b, b6r, W7, b7r, W8, b8r)
    return out.reshape(_B, 4)
